# baseline (speedup 1.0000x reference)
import functools

import jax
import jax.numpy as jnp
from jax import lax
from jax.experimental import pallas as pl
from jax.experimental.pallas import tpu as pltpu
from jax.experimental.pallas import tpu_sc as plsc

NN = 10000
EE = 320000
DD = 128
NP = 10240
NC = 2
NS = 16
NW = NC * NS
EP = 327680
EPW = EP // NW
K = 128
T = EPW // K
UC = 5
RPT = NP // NS
ZR = 80

_MESH = dict(core_axis_name="c", subcore_axis_name="s")


def _sc_hist(dst):
    @functools.partial(
        pl.kernel,
        out_type=jax.ShapeDtypeStruct((NC, NP), jnp.float32),
        mesh=plsc.VectorSubcoreMesh(**_MESH),
        scratch_types=[
            pltpu.VMEM((RPT,), jnp.float32),
            pltpu.VMEM((K,), jnp.float32),
            pltpu.VMEM((K,), jnp.int32),
            pltpu.VMEM_SHARED((NP,), jnp.float32),
        ],
    )
    def run(dst_hbm, out_hbm, zbuf, ones_v, idx_v, acc):
        cid = lax.axis_index("c")
        sid = lax.axis_index("s")
        wid = cid * NS + sid
        for j in range(RPT // 16):
            zbuf[pl.ds(j * 16, 16)] = jnp.zeros((16,), jnp.float32)
        for j in range(K // 16):
            ones_v[pl.ds(j * 16, 16)] = jnp.ones((16,), jnp.float32)
        pltpu.sync_copy(zbuf, acc.at[pl.ds(sid * RPT, RPT)])
        plsc.subcore_barrier()

        def body(i, carry):
            base = wid * EPW + i * K
            pltpu.sync_copy(dst_hbm.at[pl.ds(base, K)], idx_v)
            pltpu.sync_copy(ones_v, acc.at[idx_v], add=True)
            return carry

        lax.fori_loop(0, T, body, 0)
        plsc.subcore_barrier()
        pltpu.sync_copy(acc.at[pl.ds(sid * RPT, RPT)],
                        out_hbm.at[cid, pl.ds(sid * RPT, RPT)])

    return run(dst)


def _sc_agg(src, dst, z):
    @functools.partial(
        pl.kernel,
        out_type=jax.ShapeDtypeStruct((NC, NP, DD), jnp.float32),
        mesh=plsc.VectorSubcoreMesh(**_MESH),
        scratch_types=[
            pltpu.VMEM((ZR, DD), jnp.float32),
            pltpu.VMEM((K,), jnp.int32),
            pltpu.VMEM((K,), jnp.int32),
            pltpu.VMEM((K, DD), jnp.float32),
            pltpu.VMEM((K,), jnp.int32),
            pltpu.VMEM((K,), jnp.int32),
            pltpu.VMEM((K, DD), jnp.float32),
            pltpu.VMEM_SHARED((NP, DD), jnp.float32),
            pltpu.SemaphoreType.DMA,
            pltpu.SemaphoreType.DMA,
            pltpu.SemaphoreType.DMA,
            pltpu.SemaphoreType.DMA,
        ],
    )
    def run(src_hbm, dst_hbm, z_hbm, out_hbm, zbuf,
            isrc0, idst0, rows0, isrc1, idst1, rows1, acc,
            ga, gb, sa, sb):
        cid = lax.axis_index("c")
        sid = lax.axis_index("s")
        wid = cid * NS + sid
        isrc = [isrc0, isrc1]
        idst = [idst0, idst1]
        rows = [rows0, rows1]
        gsem = [ga, gb]
        ssem = [sa, sb]

        def zrow(i, carry):
            for j in range(DD // 16):
                zbuf[i, pl.ds(j * 16, 16)] = jnp.zeros((16,), jnp.float32)
            return carry

        lax.fori_loop(0, ZR, zrow, 0)
        for c in range(RPT // ZR):
            pltpu.sync_copy(zbuf, acc.at[pl.ds(sid * RPT + c * ZR, ZR)])
        plsc.subcore_barrier()

        def body(r, carry):
            hg = [None, None]
            hs = [None, None]
            for u in range(UC):
                b = u % 2
                if hs[b] is not None:
                    hs[b].wait()
                base = wid * EPW + (r * UC + u) * K
                pltpu.sync_copy(src_hbm.at[pl.ds(base, K)], isrc[b])
                pltpu.sync_copy(dst_hbm.at[pl.ds(base, K)], idst[b])
                hg[b] = pltpu.async_copy(z_hbm.at[isrc[b]], rows[b], gsem[b])
                if u >= 1:
                    pb = 1 - b
                    hg[pb].wait()
                    hs[pb] = pltpu.async_copy(rows[pb], acc.at[idst[pb]],
                                              ssem[pb], add=True)
            bl = (UC - 1) % 2
            hg[bl].wait()
            hs[1 - bl].wait()
            hs[bl] = pltpu.async_copy(rows[bl], acc.at[idst[bl]], ssem[bl],
                                      add=True)
            hs[bl].wait()
            return carry

        lax.fori_loop(0, T // UC, body, 0)
        plsc.subcore_barrier()
        for c in range(RPT // ZR):
            r0 = sid * RPT + c * ZR
            pltpu.sync_copy(acc.at[pl.ds(r0, ZR)], out_hbm.at[cid, pl.ds(r0, ZR)])

    return run(src, dst, z)


_RB = 2000
_GRID = NN // _RB


def _tc_head(hist_col, x, w):
    def body(h_ref, x_ref, w_ref, z_ref, d_ref):
        h = h_ref[...]
        d = lax.rsqrt(h[0] + h[1] + 1.0)
        y = jnp.dot(x_ref[...], w_ref[...], preferred_element_type=jnp.float32)
        z_ref[...] = y * d
        d_ref[...] = d

    return pl.pallas_call(
        body,
        grid=(_GRID,),
        in_specs=[
            pl.BlockSpec((NC, _RB, 1), lambda i: (0, i, 0)),
            pl.BlockSpec((_RB, DD), lambda i: (i, 0)),
            pl.BlockSpec((DD, DD), lambda i: (0, 0)),
        ],
        out_specs=(
            pl.BlockSpec((_RB, DD), lambda i: (i, 0)),
            pl.BlockSpec((_RB, 1), lambda i: (i, 0)),
        ),
        out_shape=(
            jax.ShapeDtypeStruct((NN, DD), jnp.float32),
            jax.ShapeDtypeStruct((NN, 1), jnp.float32),
        ),
    )(hist_col, x, w)


def _tc_mid(part, z1, dinv, b1, w2):
    def body(p_ref, z_ref, d_ref, b_ref, w_ref, z2_ref):
        p = p_ref[...]
        d = d_ref[...]
        h = jnp.maximum((p[0] + p[1] + z_ref[...]) * d + b_ref[...], 0.0)
        z2_ref[...] = jnp.dot(h, w_ref[...], preferred_element_type=jnp.float32) * d

    return pl.pallas_call(
        body,
        grid=(_GRID,),
        in_specs=[
            pl.BlockSpec((NC, _RB, DD), lambda i: (0, i, 0)),
            pl.BlockSpec((_RB, DD), lambda i: (i, 0)),
            pl.BlockSpec((_RB, 1), lambda i: (i, 0)),
            pl.BlockSpec((1, DD), lambda i: (0, 0)),
            pl.BlockSpec((DD, DD), lambda i: (0, 0)),
        ],
        out_specs=pl.BlockSpec((_RB, DD), lambda i: (i, 0)),
        out_shape=jax.ShapeDtypeStruct((NN, DD), jnp.float32),
    )(part, z1, dinv, b1, w2)


def _tc_tail(part, z2, dinv, b2):
    def body(p_ref, z_ref, d_ref, b_ref, o_ref):
        p = p_ref[...]
        o_ref[...] = (p[0] + p[1] + z_ref[...]) * d_ref[...] + b_ref[...]

    return pl.pallas_call(
        body,
        grid=(_GRID,),
        in_specs=[
            pl.BlockSpec((NC, _RB, DD), lambda i: (0, i, 0)),
            pl.BlockSpec((_RB, DD), lambda i: (i, 0)),
            pl.BlockSpec((_RB, 1), lambda i: (i, 0)),
            pl.BlockSpec((1, DD), lambda i: (0, 0)),
        ],
        out_specs=pl.BlockSpec((_RB, DD), lambda i: (i, 0)),
        out_shape=jax.ShapeDtypeStruct((NN, DD), jnp.float32),
    )(part, z2, dinv, b2)


def kernel(edge_index, emb, W1, b1, W2, b2):
    pad = EP - EE
    src = jnp.concatenate([edge_index[0], jnp.zeros((pad,), jnp.int32)])
    dst = jnp.concatenate([edge_index[1], jnp.full((pad,), NN, jnp.int32)])
    hist = _sc_hist(dst)
    hist_col = hist.reshape(NC, NP, 1)
    z1, dinv = _tc_head(hist_col, emb, W1)
    p1 = _sc_agg(src, dst, z1)
    z2 = _tc_mid(p1, z1, dinv, b1.reshape(1, DD), W2)
    p2 = _sc_agg(src, dst, z2)
    return _tc_tail(p2, z2, dinv, b2.reshape(1, DD))


# speedup vs baseline: 2.1922x; 2.1922x over previous
import functools

import jax
import jax.numpy as jnp
from jax import lax
from jax.experimental import pallas as pl
from jax.experimental.pallas import tpu as pltpu
from jax.experimental.pallas import tpu_sc as plsc

NN = 10000
EE = 320000
DD = 128
NP = 10240
NC = 2
NS = 16
NW = NC * NS
EPW = EE // NW
K = 80
T = EPW // K
UC = 5
RPT = NP // NS
ZR = 160

_MESH = dict(core_axis_name="c", subcore_axis_name="s")


def _sc_hist(dst):
    @functools.partial(
        pl.kernel,
        out_type=jax.ShapeDtypeStruct((NC, NP), jnp.float32),
        mesh=plsc.VectorSubcoreMesh(**_MESH),
        scratch_types=[
            pltpu.VMEM((RPT,), jnp.float32),
            pltpu.VMEM((K,), jnp.float32),
            pltpu.VMEM((K,), jnp.int32),
            pltpu.VMEM_SHARED((NP,), jnp.float32),
        ],
    )
    def run(dst_hbm, out_hbm, zbuf, ones_v, idx_v, acc):
        cid = lax.axis_index("c")
        sid = lax.axis_index("s")
        wid = cid * NS + sid
        for j in range(RPT // 16):
            zbuf[pl.ds(j * 16, 16)] = jnp.zeros((16,), jnp.float32)
        for j in range(K // 16):
            ones_v[pl.ds(j * 16, 16)] = jnp.ones((16,), jnp.float32)
        pltpu.sync_copy(zbuf, acc.at[pl.ds(sid * RPT, RPT)])
        plsc.subcore_barrier()

        def body(i, carry):
            base = wid * EPW + i * K
            pltpu.sync_copy(dst_hbm.at[pl.ds(base, K)], idx_v)
            pltpu.sync_copy(ones_v, acc.at[idx_v], add=True)
            return carry

        lax.fori_loop(0, T, body, 0)
        plsc.subcore_barrier()
        pltpu.sync_copy(acc.at[pl.ds(sid * RPT, RPT)],
                        out_hbm.at[cid, pl.ds(sid * RPT, RPT)])

    return run(dst)


def _sc_agg(src, dst, z):
    @functools.partial(
        pl.kernel,
        out_type=jax.ShapeDtypeStruct((NC, NP, DD), jnp.float32),
        mesh=plsc.VectorSubcoreMesh(**_MESH),
        scratch_types=[
            pltpu.VMEM((ZR, DD), jnp.float32),
            pltpu.VMEM((K,), jnp.int32),
            pltpu.VMEM((K,), jnp.int32),
            pltpu.VMEM((K, DD), jnp.float32),
            pltpu.VMEM((K,), jnp.int32),
            pltpu.VMEM((K,), jnp.int32),
            pltpu.VMEM((K, DD), jnp.float32),
            pltpu.VMEM_SHARED((NP, DD), jnp.float32),
            pltpu.SemaphoreType.DMA,
            pltpu.SemaphoreType.DMA,
            pltpu.SemaphoreType.DMA,
            pltpu.SemaphoreType.DMA,
        ],
    )
    def run(src_hbm, dst_hbm, z_hbm, out_hbm, zbuf,
            isrc0, idst0, rows0, isrc1, idst1, rows1, acc,
            ga, gb, sa, sb):
        cid = lax.axis_index("c")
        sid = lax.axis_index("s")
        wid = cid * NS + sid
        isrc = [isrc0, isrc1]
        idst = [idst0, idst1]
        rows = [rows0, rows1]
        gsem = [ga, gb]
        ssem = [sa, sb]

        def zrow(i, carry):
            for j in range(DD // 16):
                zbuf[i, pl.ds(j * 16, 16)] = jnp.zeros((16,), jnp.float32)
            return carry

        lax.fori_loop(0, ZR, zrow, 0)
        for c in range(RPT // ZR):
            pltpu.sync_copy(zbuf, acc.at[pl.ds(sid * RPT + c * ZR, ZR)])
        plsc.subcore_barrier()

        def body(r, carry):
            hg = [None, None]
            hs = [None, None]
            for u in range(UC):
                b = u % 2
                if hs[b] is not None:
                    hs[b].wait()
                base = wid * EPW + (r * UC + u) * K
                pltpu.sync_copy(src_hbm.at[pl.ds(base, K)], isrc[b])
                pltpu.sync_copy(dst_hbm.at[pl.ds(base, K)], idst[b])
                hg[b] = pltpu.async_copy(z_hbm.at[isrc[b]], rows[b], gsem[b])
                if u >= 1:
                    pb = 1 - b
                    hg[pb].wait()
                    hs[pb] = pltpu.async_copy(rows[pb], acc.at[idst[pb]],
                                              ssem[pb], add=True)
            bl = (UC - 1) % 2
            hg[bl].wait()
            hs[1 - bl].wait()
            hs[bl] = pltpu.async_copy(rows[bl], acc.at[idst[bl]], ssem[bl],
                                      add=True)
            hs[bl].wait()
            return carry

        lax.fori_loop(0, T // UC, body, 0)
        plsc.subcore_barrier()
        for c in range(RPT // ZR):
            r0 = sid * RPT + c * ZR
            pltpu.sync_copy(acc.at[pl.ds(r0, ZR)], out_hbm.at[cid, pl.ds(r0, ZR)])

    return run(src, dst, z)


_RB = 2000
_GRID = NN // _RB


def _tc_head(hist_col, x, w):
    def body(h_ref, x_ref, w_ref, z_ref, d_ref):
        h = h_ref[...]
        d = lax.rsqrt(h[0] + h[1] + 1.0)
        y = jnp.dot(x_ref[...], w_ref[...], preferred_element_type=jnp.float32)
        z_ref[...] = y * d
        d_ref[...] = d

    return pl.pallas_call(
        body,
        grid=(_GRID,),
        in_specs=[
            pl.BlockSpec((NC, _RB, 1), lambda i: (0, i, 0)),
            pl.BlockSpec((_RB, DD), lambda i: (i, 0)),
            pl.BlockSpec((DD, DD), lambda i: (0, 0)),
        ],
        out_specs=(
            pl.BlockSpec((_RB, DD), lambda i: (i, 0)),
            pl.BlockSpec((_RB, 1), lambda i: (i, 0)),
        ),
        out_shape=(
            jax.ShapeDtypeStruct((NN, DD), jnp.float32),
            jax.ShapeDtypeStruct((NN, 1), jnp.float32),
        ),
    )(hist_col, x, w)


def _tc_mid(part, z1, dinv, b1, w2):
    def body(p_ref, z_ref, d_ref, b_ref, w_ref, z2_ref):
        p = p_ref[...]
        d = d_ref[...]
        h = jnp.maximum((p[0] + p[1] + z_ref[...]) * d + b_ref[...], 0.0)
        z2_ref[...] = jnp.dot(h, w_ref[...], preferred_element_type=jnp.float32) * d

    return pl.pallas_call(
        body,
        grid=(_GRID,),
        in_specs=[
            pl.BlockSpec((NC, _RB, DD), lambda i: (0, i, 0)),
            pl.BlockSpec((_RB, DD), lambda i: (i, 0)),
            pl.BlockSpec((_RB, 1), lambda i: (i, 0)),
            pl.BlockSpec((1, DD), lambda i: (0, 0)),
            pl.BlockSpec((DD, DD), lambda i: (0, 0)),
        ],
        out_specs=pl.BlockSpec((_RB, DD), lambda i: (i, 0)),
        out_shape=jax.ShapeDtypeStruct((NN, DD), jnp.float32),
    )(part, z1, dinv, b1, w2)


def _tc_tail(part, z2, dinv, b2):
    def body(p_ref, z_ref, d_ref, b_ref, o_ref):
        p = p_ref[...]
        o_ref[...] = (p[0] + p[1] + z_ref[...]) * d_ref[...] + b_ref[...]

    return pl.pallas_call(
        body,
        grid=(_GRID,),
        in_specs=[
            pl.BlockSpec((NC, _RB, DD), lambda i: (0, i, 0)),
            pl.BlockSpec((_RB, DD), lambda i: (i, 0)),
            pl.BlockSpec((_RB, 1), lambda i: (i, 0)),
            pl.BlockSpec((1, DD), lambda i: (0, 0)),
        ],
        out_specs=pl.BlockSpec((_RB, DD), lambda i: (i, 0)),
        out_shape=jax.ShapeDtypeStruct((NN, DD), jnp.float32),
    )(part, z2, dinv, b2)


def kernel(edge_index, emb, W1, b1, W2, b2):
    src = edge_index[0]
    dst = edge_index[1]
    hist = _sc_hist(dst)
    hist_col = hist.reshape(NC, NP, 1)
    z1, dinv = _tc_head(hist_col, emb, W1)
    p1 = _sc_agg(src, dst, z1)
    z2 = _tc_mid(p1, z1, dinv, b1.reshape(1, DD), W2)
    p2 = _sc_agg(src, dst, z2)
    return _tc_tail(p2, z2, dinv, b2.reshape(1, DD))


# speedup vs baseline: 2.2280x; 1.0163x over previous
import functools

import jax
import jax.numpy as jnp
from jax import lax
from jax.experimental import pallas as pl
from jax.experimental.pallas import tpu as pltpu
from jax.experimental.pallas import tpu_sc as plsc

NN = 10000
EE = 320000
DD = 128
NP = 10240
NC = 2
NS = 16
NW = NC * NS
EPW = EE // NW
K = 80
T = EPW // K
UC = 5
RPT = NP // NS
ZR = 160

_MESH = dict(core_axis_name="c", subcore_axis_name="s")


def _sc_hist(dst):
    @functools.partial(
        pl.kernel,
        out_type=jax.ShapeDtypeStruct((NC, NP), jnp.float32),
        mesh=plsc.VectorSubcoreMesh(**_MESH),
        scratch_types=[
            pltpu.VMEM((RPT,), jnp.float32),
            pltpu.VMEM((K,), jnp.float32),
            pltpu.VMEM((K,), jnp.int32),
            pltpu.VMEM((K,), jnp.int32),
            pltpu.VMEM_SHARED((NP,), jnp.float32),
            pltpu.SemaphoreType.DMA,
            pltpu.SemaphoreType.DMA,
        ],
    )
    def run(dst_hbm, out_hbm, zbuf, ones_v, idx0, idx1, acc, sa, sb):
        cid = lax.axis_index("c")
        sid = lax.axis_index("s")
        wid = cid * NS + sid
        idx = [idx0, idx1]
        ssem = [sa, sb]
        for j in range(RPT // 16):
            zbuf[pl.ds(j * 16, 16)] = jnp.zeros((16,), jnp.float32)
        for j in range(K // 16):
            ones_v[pl.ds(j * 16, 16)] = jnp.ones((16,), jnp.float32)
        pltpu.sync_copy(zbuf, acc.at[pl.ds(sid * RPT, RPT)])
        plsc.subcore_barrier()

        def body(r, carry):
            hs = [None, None]
            for u in range(UC):
                b = u % 2
                if hs[b] is not None:
                    hs[b].wait()
                base = wid * EPW + (r * UC + u) * K
                pltpu.sync_copy(dst_hbm.at[pl.ds(base, K)], idx[b])
                hs[b] = pltpu.async_copy(ones_v, acc.at[idx[b]], ssem[b],
                                         add=True)
            hs[0].wait()
            hs[1].wait()
            return carry

        lax.fori_loop(0, T // UC, body, 0)
        plsc.subcore_barrier()
        pltpu.sync_copy(acc.at[pl.ds(sid * RPT, RPT)],
                        out_hbm.at[cid, pl.ds(sid * RPT, RPT)])

    return run(dst)


def _sc_agg(src, dst, z):
    @functools.partial(
        pl.kernel,
        out_type=jax.ShapeDtypeStruct((NC, NP, DD), jnp.float32),
        mesh=plsc.VectorSubcoreMesh(**_MESH),
        scratch_types=[
            pltpu.VMEM((ZR, DD), jnp.float32),
            pltpu.VMEM((K,), jnp.int32),
            pltpu.VMEM((K,), jnp.int32),
            pltpu.VMEM((K, DD), jnp.float32),
            pltpu.VMEM((K,), jnp.int32),
            pltpu.VMEM((K,), jnp.int32),
            pltpu.VMEM((K, DD), jnp.float32),
            pltpu.VMEM_SHARED((NP, DD), jnp.float32),
            pltpu.SemaphoreType.DMA,
            pltpu.SemaphoreType.DMA,
            pltpu.SemaphoreType.DMA,
            pltpu.SemaphoreType.DMA,
        ],
    )
    def run(src_hbm, dst_hbm, z_hbm, out_hbm, zbuf,
            isrc0, idst0, rows0, isrc1, idst1, rows1, acc,
            ga, gb, sa, sb):
        cid = lax.axis_index("c")
        sid = lax.axis_index("s")
        wid = cid * NS + sid
        isrc = [isrc0, isrc1]
        idst = [idst0, idst1]
        rows = [rows0, rows1]
        gsem = [ga, gb]
        ssem = [sa, sb]

        def zrow(i, carry):
            for j in range(DD // 16):
                zbuf[i, pl.ds(j * 16, 16)] = jnp.zeros((16,), jnp.float32)
            return carry

        lax.fori_loop(0, ZR, zrow, 0)
        for c in range(RPT // ZR):
            pltpu.sync_copy(zbuf, acc.at[pl.ds(sid * RPT + c * ZR, ZR)])
        plsc.subcore_barrier()

        def body(r, carry):
            hg = [None, None]
            hs = [None, None]
            for u in range(UC):
                b = u % 2
                if hs[b] is not None:
                    hs[b].wait()
                base = wid * EPW + (r * UC + u) * K
                pltpu.sync_copy(src_hbm.at[pl.ds(base, K)], isrc[b])
                pltpu.sync_copy(dst_hbm.at[pl.ds(base, K)], idst[b])
                hg[b] = pltpu.async_copy(z_hbm.at[isrc[b]], rows[b], gsem[b])
                if u >= 1:
                    pb = 1 - b
                    hg[pb].wait()
                    hs[pb] = pltpu.async_copy(rows[pb], acc.at[idst[pb]],
                                              ssem[pb], add=True)
            bl = (UC - 1) % 2
            hg[bl].wait()
            hs[1 - bl].wait()
            hs[bl] = pltpu.async_copy(rows[bl], acc.at[idst[bl]], ssem[bl],
                                      add=True)
            hs[bl].wait()
            return carry

        lax.fori_loop(0, T // UC, body, 0)
        plsc.subcore_barrier()
        for c in range(RPT // ZR):
            r0 = sid * RPT + c * ZR
            pltpu.sync_copy(acc.at[pl.ds(r0, ZR)], out_hbm.at[cid, pl.ds(r0, ZR)])

    return run(src, dst, z)


_RB = 2000
_GRID = NN // _RB


def _tc_head(hist_col, x, w):
    def body(h_ref, x_ref, w_ref, z_ref, d_ref):
        h = h_ref[...]
        d = lax.rsqrt(h[0] + h[1] + 1.0)
        y = jnp.dot(x_ref[...], w_ref[...], preferred_element_type=jnp.float32)
        z_ref[...] = y * d
        d_ref[...] = d

    return pl.pallas_call(
        body,
        grid=(_GRID,),
        in_specs=[
            pl.BlockSpec((NC, _RB, 1), lambda i: (0, i, 0)),
            pl.BlockSpec((_RB, DD), lambda i: (i, 0)),
            pl.BlockSpec((DD, DD), lambda i: (0, 0)),
        ],
        out_specs=(
            pl.BlockSpec((_RB, DD), lambda i: (i, 0)),
            pl.BlockSpec((_RB, 1), lambda i: (i, 0)),
        ),
        out_shape=(
            jax.ShapeDtypeStruct((NN, DD), jnp.float32),
            jax.ShapeDtypeStruct((NN, 1), jnp.float32),
        ),
    )(hist_col, x, w)


def _tc_mid(part, z1, dinv, b1, w2):
    def body(p_ref, z_ref, d_ref, b_ref, w_ref, z2_ref):
        p = p_ref[...]
        d = d_ref[...]
        h = jnp.maximum((p[0] + p[1] + z_ref[...]) * d + b_ref[...], 0.0)
        z2_ref[...] = jnp.dot(h, w_ref[...], preferred_element_type=jnp.float32) * d

    return pl.pallas_call(
        body,
        grid=(_GRID,),
        in_specs=[
            pl.BlockSpec((NC, _RB, DD), lambda i: (0, i, 0)),
            pl.BlockSpec((_RB, DD), lambda i: (i, 0)),
            pl.BlockSpec((_RB, 1), lambda i: (i, 0)),
            pl.BlockSpec((1, DD), lambda i: (0, 0)),
            pl.BlockSpec((DD, DD), lambda i: (0, 0)),
        ],
        out_specs=pl.BlockSpec((_RB, DD), lambda i: (i, 0)),
        out_shape=jax.ShapeDtypeStruct((NN, DD), jnp.float32),
    )(part, z1, dinv, b1, w2)


def _tc_tail(part, z2, dinv, b2):
    def body(p_ref, z_ref, d_ref, b_ref, o_ref):
        p = p_ref[...]
        o_ref[...] = (p[0] + p[1] + z_ref[...]) * d_ref[...] + b_ref[...]

    return pl.pallas_call(
        body,
        grid=(_GRID,),
        in_specs=[
            pl.BlockSpec((NC, _RB, DD), lambda i: (0, i, 0)),
            pl.BlockSpec((_RB, DD), lambda i: (i, 0)),
            pl.BlockSpec((_RB, 1), lambda i: (i, 0)),
            pl.BlockSpec((1, DD), lambda i: (0, 0)),
        ],
        out_specs=pl.BlockSpec((_RB, DD), lambda i: (i, 0)),
        out_shape=jax.ShapeDtypeStruct((NN, DD), jnp.float32),
    )(part, z2, dinv, b2)


def kernel(edge_index, emb, W1, b1, W2, b2):
    src = edge_index[0]
    dst = edge_index[1]
    hist = _sc_hist(dst)
    hist_col = hist.reshape(NC, NP, 1)
    z1, dinv = _tc_head(hist_col, emb, W1)
    p1 = _sc_agg(src, dst, z1)
    z2 = _tc_mid(p1, z1, dinv, b1.reshape(1, DD), W2)
    p2 = _sc_agg(src, dst, z2)
    return _tc_tail(p2, z2, dinv, b2.reshape(1, DD))
